# baseline (speedup 1.0000x reference)
import functools

import jax
import jax.numpy as jnp
from jax import lax
from jax.experimental import pallas as pl
from jax.experimental.pallas import tpu as pltpu
from jax.experimental.pallas import tpu_sc as plsc

N = 100000
D = 128
OUT = 128
G = 160
NOFF = 27
NOFF_PAD = 32

NW = 32
NPAD = 100352
NCHUNK = 2
PC = NPAD // NCHUNK
P_PER_W = PC // NW
BLK = 1024
NBLK = NPAD // BLK
NBLK_C = PC // BLK
SX = 10
SY = 8


def _pack_zy_body(g_ref, r_ref):
    p = (g_ref[...] != 0.0).astype(jnp.int32)
    q = jnp.roll(p, 1, 2) | (p << 1) | (jnp.roll(p, -1, 2) << 2)
    r_ref[...] = jnp.roll(q, 1, 1) | (q << 3) | (jnp.roll(q, -1, 1) << 6)


def _pack_x_body(r_ref, b_ref):
    r = r_ref[...]
    b_ref[...] = jnp.roll(r, 1, 0) | (r << 9) | (jnp.roll(r, -1, 0) << 18)


def _sc_masks(voxflat, bgrid_flat, chunk):
    mesh = plsc.VectorSubcoreMesh(core_axis_name="c", subcore_axis_name="s")

    @functools.partial(
        pl.kernel,
        mesh=mesh,
        out_type=jax.ShapeDtypeStruct((PC * NOFF_PAD,), jnp.float32),
        scratch_types=[
            pltpu.VMEM((P_PER_W * 3,), jnp.int32),
            pltpu.VMEM((P_PER_W,), jnp.int32),
            pltpu.VMEM((P_PER_W,), jnp.int32),
            pltpu.VMEM((P_PER_W * NOFF_PAD,), jnp.float32),
            pltpu.SemaphoreType.DMA,
        ],
        compiler_params=pltpu.CompilerParams(needs_layout_passes=False),
    )
    def k(vox_hbm, bgrid_hbm, m_hbm, vox_v, idx_v, bits_v, m_v, sem):
        wid = lax.axis_index("s") * 2 + lax.axis_index("c")
        base = pl.multiple_of(wid * P_PER_W, 16)
        gbase = pl.multiple_of(chunk * PC + wid * P_PER_W, 16)
        pltpu.sync_copy(vox_hbm.at[pl.ds(gbase * 3, P_PER_W * 3)], vox_v)
        lanes = lax.iota(jnp.int32, 16)

        def idx_body(vi, carry):
            s = pl.multiple_of(vi * 16, 16)
            pos = (lanes + s) * 3
            x = plsc.load_gather(vox_v, [pos])
            y = plsc.load_gather(vox_v, [pos + 1])
            z = plsc.load_gather(vox_v, [pos + 2])
            idx_v[pl.ds(s, 16)] = (x * G + y) * G + z
            return carry

        lax.fori_loop(0, P_PER_W // 16, idx_body, 0)
        pltpu.async_copy(bgrid_hbm.at[idx_v], bits_v, sem).wait()

        def unpack_body(vi, carry):
            s = pl.multiple_of(vi * 16, 16)
            b = bits_v[pl.ds(s, 16)]
            mpos = (lanes + s) * NOFF_PAD
            for o in range(NOFF):
                mo = ((b >> o) & 1).astype(jnp.float32)
                plsc.store_scatter(m_v, [mpos + o], mo)
            return carry

        lax.fori_loop(0, P_PER_W // 16, unpack_body, 0)
        pltpu.sync_copy(
            m_v, m_hbm.at[pl.ds(base * NOFF_PAD, P_PER_W * NOFF_PAD)])

    return k(voxflat, bgrid_flat)


def _tc_compute(x_ref, m_ref, w_ref, o_ref):
    y = jnp.dot(x_ref[...].astype(jnp.bfloat16), w_ref[...],
                preferred_element_type=jnp.float32)
    m = (m_ref[...] != 0.0).astype(jnp.float32)
    acc = jnp.zeros((BLK, OUT), jnp.float32)
    for o in range(NOFF):
        acc = acc + m[:, o:o + 1] * y[:, o * OUT:(o + 1) * OUT]
    o_ref[...] = jnp.maximum(acc, 0.0)


def _tc_body_first(x_ref, m_ref, w_ref, o_ref):
    _tc_compute(x_ref, m_ref, w_ref, o_ref)


def _tc_body_next(x_ref, m_ref, w_ref, prev_ref, o_ref):
    del prev_ref
    _tc_compute(x_ref, m_ref, w_ref, o_ref)


def kernel(inputs, voxel_idx, pts_per_voxel_inv, conv_w):
    voxflat = jnp.pad(voxel_idx.reshape(N * 3), (0, (NPAD - N) * 3))
    grid3 = pts_per_voxel_inv.reshape(G, G, G)
    w_bf = (conv_w.reshape(NOFF, D, OUT)
            .transpose(1, 0, 2)
            .reshape(D, NOFF * OUT)
            .astype(jnp.bfloat16))

    r = pl.pallas_call(
        _pack_zy_body,
        grid=(G // SX,),
        in_specs=[pl.BlockSpec((SX, G, G), lambda i: (i, 0, 0))],
        out_specs=pl.BlockSpec((SX, G, G), lambda i: (i, 0, 0)),
        out_shape=jax.ShapeDtypeStruct((G, G, G), jnp.int32),
    )(grid3)
    bgrid = pl.pallas_call(
        _pack_x_body,
        grid=(G // SY,),
        in_specs=[pl.BlockSpec((G, SY, G), lambda i: (0, i, 0))],
        out_specs=pl.BlockSpec((G, SY, G), lambda i: (0, i, 0)),
        out_shape=jax.ShapeDtypeStruct((G, G, G), jnp.int32),
    )(r)

    bgrid_flat = bgrid.reshape(G * G * G)

    masks = [_sc_masks(voxflat, bgrid_flat, c).reshape(PC, NOFF_PAD)
             for c in range(NCHUNK)]

    out = None
    for c in range(NCHUNK):
        off = c * NBLK_C
        in_specs = [
            pl.BlockSpec((BLK, D), lambda i, off=off: (i + off, 0)),
            pl.BlockSpec((BLK, NOFF_PAD), lambda i: (i, 0)),
            pl.BlockSpec((D, NOFF * OUT), lambda i: (0, 0)),
        ]
        if c == 0:
            out = pl.pallas_call(
                _tc_body_first,
                grid=(NBLK_C,),
                in_specs=in_specs,
                out_specs=pl.BlockSpec((BLK, OUT),
                                       lambda i, off=off: (i + off, 0)),
                out_shape=jax.ShapeDtypeStruct((N, OUT), jnp.float32),
            )(inputs, masks[c], w_bf)
        else:
            out = pl.pallas_call(
                _tc_body_next,
                grid=(NBLK_C,),
                in_specs=in_specs + [pl.BlockSpec(memory_space=pl.ANY)],
                out_specs=pl.BlockSpec((BLK, OUT),
                                       lambda i, off=off: (i + off, 0)),
                out_shape=jax.ShapeDtypeStruct((N, OUT), jnp.float32),
                input_output_aliases={3: 0},
            )(inputs, masks[c], w_bf, out)
    return out
